# baseline (speedup 1.0000x reference)
import math

import jax
import jax.numpy as jnp
from jax.experimental import pallas as pl
from jax.experimental.pallas import tpu as pltpu

_WEIGHT = 2e-9
_SIGMA_RGB = 0.15
_SIGMA_XY = 100.0
_SCALE = 0.5
_OH, _OW = 64, 64
_P = _OH * _OW
_KP = 24
_TI = 512
_LOG2E = 1.4426950408889634


def _crf_tile(feat_i_ref, featT_ref, seg_ref, seg_i_ref, out_ref):
    fi = feat_i_ref[0]
    ft = featT_ref[0]
    arg = jax.lax.dot_general(
        fi, ft, (((1,), (0,)), ((), ())),
        preferred_element_type=jnp.float32)
    wk = jnp.exp2(jnp.minimum(arg, 0.0))
    filt = jax.lax.dot_general(
        wk, seg_ref[0], (((1,), (0,)), ((), ())),
        preferred_element_type=jnp.float32)
    out_ref[0, 0, :] = jnp.sum(seg_i_ref[0] * filt, axis=0)


def kernel(images, segmentations, ROIs):
    n_img, _, h, w = images.shape
    k_cls = segmentations.shape[1]
    ni = _P // _TI

    img_s = images[:, :, ::2, ::2]
    roi_s = ROIs[:, ::2, ::2]
    s00 = segmentations[:, :, ::2, ::2]
    s01 = segmentations[:, :, ::2, 1::2]
    s10 = segmentations[:, :, 1::2, ::2]
    s11 = segmentations[:, :, 1::2, 1::2]
    seg_s = 0.5 * (0.5 * (s00 + s01) + 0.5 * (s10 + s11))
    seg_m = seg_s * roi_s[:, None]

    sxy = _SIGMA_XY * _SCALE
    rt = math.sqrt(_LOG2E)
    yy, xx = jnp.meshgrid(jnp.arange(_OH, dtype=jnp.float32),
                          jnp.arange(_OW, dtype=jnp.float32), indexing="ij")
    px = (xx.reshape(-1) * (rt / sxy))[None, :, None]
    py = (yy.reshape(-1) * (rt / sxy))[None, :, None]
    img_f = img_s.reshape(n_img, 3, _P).transpose(0, 2, 1) * (rt / _SIGMA_RGB)
    ax = jnp.concatenate([
        jnp.broadcast_to(px, (n_img, _P, 1)),
        jnp.broadcast_to(py, (n_img, _P, 1)),
        img_f,
    ], axis=2)
    m = -0.5 * jnp.sum(ax * ax, axis=2, keepdims=True)
    one = jnp.ones((n_img, _P, 1), jnp.float32)
    zero = jnp.zeros((n_img, _P, 1), jnp.float32)
    feat = jnp.concatenate([ax, m, one, zero], axis=2)
    feat_r = jnp.concatenate([ax, one, m, zero], axis=2)
    featT = feat_r.transpose(0, 2, 1)

    seg_f = seg_m.reshape(n_img, k_cls, _P).transpose(0, 2, 1)
    seg_p = jnp.pad(seg_f, ((0, 0), (0, 0), (0, _KP - k_cls)))

    grid = (n_img * ni,)
    partials = pl.pallas_call(
        _crf_tile,
        grid=grid,
        in_specs=[
            pl.BlockSpec((1, _TI, 8), lambda p: (p // ni, p % ni, 0)),
            pl.BlockSpec((1, 8, _P), lambda p: (p // ni, 0, 0)),
            pl.BlockSpec((1, _P, _KP), lambda p: (p // ni, 0, 0)),
            pl.BlockSpec((1, _TI, _KP), lambda p: (p // ni, p % ni, 0)),
        ],
        out_specs=pl.BlockSpec((1, 1, _KP), lambda p: (p, 0, 0)),
        out_shape=jax.ShapeDtypeStruct((n_img * ni, 1, _KP), jnp.float32),
        compiler_params=pltpu.CompilerParams(
            dimension_semantics=("parallel",),
            vmem_limit_bytes=100 * 1024 * 1024,
        ),
    )(feat, featT, seg_p, seg_p)

    return (-_WEIGHT / n_img) * jnp.sum(partials)


# speedup vs baseline: 1.0244x; 1.0244x over previous
import math

import jax
import jax.numpy as jnp
from jax.experimental import pallas as pl
from jax.experimental.pallas import tpu as pltpu

_WEIGHT = 2e-9
_SIGMA_RGB = 0.15
_SIGMA_XY = 100.0
_SCALE = 0.5
_OH, _OW = 64, 64
_P = _OH * _OW
_KP = 24
_TI = 512
_LOG2E = 1.4426950408889634


def _crf_tile(featL_ref, featR_ref, seg_ref, seg_i_ref, out_ref):
    fl = featL_ref[0]
    fr = featR_ref[0]
    arg = jax.lax.dot_general(
        fl, fr, (((0,), (0,)), ((), ())),
        preferred_element_type=jnp.float32)
    wk = jnp.exp2(jnp.minimum(arg, 0.0))
    filt = jax.lax.dot_general(
        wk, seg_ref[0], (((1,), (1,)), ((), ())),
        preferred_element_type=jnp.float32)
    prod = jax.lax.dot_general(
        seg_i_ref[0], filt, (((1,), (0,)), ((), ())),
        preferred_element_type=jnp.float32)
    r_ix = jax.lax.broadcasted_iota(jnp.int32, (_KP, _KP), 0)
    c_ix = jax.lax.broadcasted_iota(jnp.int32, (_KP, _KP), 1)
    diag = jnp.where(r_ix == c_ix, prod, 0.0)
    out_ref[0, 0, :] = jnp.sum(diag, axis=0)


def kernel(images, segmentations, ROIs):
    n_img, _, h, w = images.shape
    k_cls = segmentations.shape[1]
    ni = _P // _TI

    img_s = images[:, :, ::2, ::2]
    roi_s = ROIs[:, ::2, ::2]
    s00 = segmentations[:, :, ::2, ::2]
    s01 = segmentations[:, :, ::2, 1::2]
    s10 = segmentations[:, :, 1::2, ::2]
    s11 = segmentations[:, :, 1::2, 1::2]
    seg_s = 0.5 * (0.5 * (s00 + s01) + 0.5 * (s10 + s11))
    seg_m = seg_s * roi_s[:, None]

    sxy = _SIGMA_XY * _SCALE
    rt = math.sqrt(_LOG2E)
    yy, xx = jnp.meshgrid(jnp.arange(_OH, dtype=jnp.float32),
                          jnp.arange(_OW, dtype=jnp.float32), indexing="ij")
    px = (xx.reshape(-1) * (rt / sxy))[None, None, :]
    py = (yy.reshape(-1) * (rt / sxy))[None, None, :]
    img_f = img_s.reshape(n_img, 3, _P) * (rt / _SIGMA_RGB)
    ax = jnp.concatenate([
        jnp.broadcast_to(px, (n_img, 1, _P)),
        jnp.broadcast_to(py, (n_img, 1, _P)),
        img_f,
    ], axis=1)
    m = -0.5 * jnp.sum(ax * ax, axis=1, keepdims=True)
    one = jnp.ones((n_img, 1, _P), jnp.float32)
    zero = jnp.zeros((n_img, 1, _P), jnp.float32)
    feat_l = jnp.concatenate([ax, m, one, zero], axis=1)
    feat_r = jnp.concatenate([ax, one, m, zero], axis=1)

    seg_f = seg_m.reshape(n_img, k_cls, _P)
    seg_p = jnp.pad(seg_f, ((0, 0), (0, _KP - k_cls), (0, 0)))

    grid = (n_img * ni,)
    partials = pl.pallas_call(
        _crf_tile,
        grid=grid,
        in_specs=[
            pl.BlockSpec((1, 8, _TI), lambda p: (p // ni, 0, p % ni)),
            pl.BlockSpec((1, 8, _P), lambda p: (p // ni, 0, 0)),
            pl.BlockSpec((1, _KP, _P), lambda p: (p // ni, 0, 0)),
            pl.BlockSpec((1, _KP, _TI), lambda p: (p // ni, 0, p % ni)),
        ],
        out_specs=pl.BlockSpec((1, 1, _KP), lambda p: (p, 0, 0)),
        out_shape=jax.ShapeDtypeStruct((n_img * ni, 1, _KP), jnp.float32),
        compiler_params=pltpu.CompilerParams(
            dimension_semantics=("parallel",),
            vmem_limit_bytes=100 * 1024 * 1024,
        ),
    )(feat_l, feat_r, seg_p, seg_p)

    return (-_WEIGHT / n_img) * jnp.sum(partials)
